# baseline (speedup 1.0000x reference)
import functools

import jax
import jax.numpy as jnp
from jax import lax
from jax.experimental import pallas as pl
from jax.experimental.pallas import tpu as pltpu
from jax.experimental.pallas import tpu_sc as plsc

GROUP = 128
NUM_CORES = 2
NUM_SUBCORES = 16
NUM_WORKERS = NUM_CORES * NUM_SUBCORES
LANES = 16
VPAD = 32


@functools.lru_cache(maxsize=None)
def _build_sc_kernel(S0, S1, V, D):
    rows_per_w = S0 // NUM_WORKERS
    groups_per_w = rows_per_w * (S1 // GROUP)
    slab = S1 * D
    chunks = S1 // LANES
    cpg = GROUP // LANES
    mesh = plsc.VectorSubcoreMesh(
        core_axis_name="c", subcore_axis_name="s",
        num_cores=NUM_CORES, num_subcores=NUM_SUBCORES)

    @functools.partial(
        pl.kernel,
        out_type=jax.ShapeDtypeStruct((S0, slab), jnp.float32),
        mesh=mesh,
        compiler_params=pltpu.CompilerParams(
            needs_layout_passes=False, use_tc_tiling_on_sc=False),
        scratch_types=(
            pltpu.VMEM((groups_per_w, GROUP), jnp.int32),
            pltpu.VMEM((VPAD, D), jnp.float32),
            pltpu.VMEM((D * VPAD,), jnp.float32),
            pltpu.VMEM((slab,), jnp.float32),
            pltpu.VMEM((slab,), jnp.float32),
            pltpu.SemaphoreType.DMA,
            pltpu.SemaphoreType.DMA,
        ),
    )
    def sc_fill(table_hbm, idx_hbm, out_hbm, idx_v, table_v, tableT, buf0,
                buf1, sem0, sem1):
        sid = lax.axis_index("s")
        wid = sid * NUM_CORES + lax.axis_index("c")
        i0 = wid * rows_per_w

        pltpu.sync_copy(idx_hbm.at[pl.ds(wid * groups_per_w, groups_per_w)],
                        idx_v)

        pltpu.sync_copy(table_hbm, table_v.at[pl.ds(0, V)])
        diag0 = lax.iota(jnp.int32, LANES)
        diag1 = diag0 + LANES
        ones = jnp.ones((LANES,), jnp.float32)
        plsc.addupdate_scatter(table_v, [diag0, diag0], ones)
        plsc.addupdate_scatter(table_v, [diag1, diag1], ones,
                               mask=diag1 < V)

        iota16 = lax.iota(jnp.int32, LANES)
        for d in range(D):
            dvec = jnp.full((LANES,), d, jnp.int32)
            v1 = plsc.load_gather(table_v, [iota16, dvec])
            tableT[pl.ds(VPAD * d, LANES)] = v1
            v2 = plsc.load_gather(table_v, [iota16 + LANES, dvec])
            tableT[pl.ds(VPAD * d + LANES, LANES)] = v2

        bufs = (buf0, buf1)
        sems = (sem0, sem1)

        def fill_row(ii, buf):
            ilv = 8

            def chunk_body(cg, _):
                toks = []
                tjs = []
                for u in range(ilv):
                    ch = cg * ilv + u
                    jt = ch // cpg
                    qq = ch - jt * cpg
                    toks.append(idx_v[(S1 // GROUP) * ii + jt,
                                      pl.dslice(qq * LANES, LANES)])
                    tjs.append(jt * (8 * GROUP) + qq * LANES)
                for d in range(D):
                    off = (d // 8) * ((S1 // GROUP) * 8 * GROUP) \
                        + (d % 8) * GROUP
                    vs = [plsc.load_gather(
                              tableT.at[pl.ds(VPAD * d, VPAD)], [toks[u]])
                          for u in range(ilv)]
                    for u in range(ilv):
                        buf[pl.ds(tjs[u] + off, LANES)] = vs[u]
                return 0

            lax.fori_loop(0, chunks // ilv, chunk_body, 0)

        def wb(ii, buf, sem):
            return pltpu.make_async_copy(buf, out_hbm.at[i0 + ii], sem)

        def row_pair(p, _):
            ii_a = p * 2
            ii_b = ii_a + 1

            @pl.when(p > 0)
            def _():
                wb(ii_a - 2, buf0, sem0).wait()
            fill_row(ii_a, buf0)
            wb(ii_a, buf0, sem0).start()

            @pl.when(p > 0)
            def _():
                wb(ii_b - 2, buf1, sem1).wait()
            fill_row(ii_b, buf1)
            wb(ii_b, buf1, sem1).start()
            return 0

        lax.fori_loop(0, rows_per_w // 2, row_pair, 0)
        wb(rows_per_w - 2, buf0, sem0).wait()
        wb(rows_per_w - 1, buf1, sem1).wait()

    return sc_fill


def kernel(input, emb_table):
    S0, S1 = input.shape
    V, D = emb_table.shape
    idx2d = input.reshape((S0 * S1) // GROUP, GROUP)
    out = _build_sc_kernel(S0, S1, V, D)(emb_table, idx2d)
    out = out.reshape(S0, D // 8, S1 // GROUP, 8, GROUP)
    out = out.transpose(0, 2, 4, 1, 3).reshape(S0, S1, D)
    return out


# speedup vs baseline: 1.0612x; 1.0612x over previous
import functools

import jax
import jax.numpy as jnp
from jax import lax
from jax.experimental import pallas as pl
from jax.experimental.pallas import tpu as pltpu
from jax.experimental.pallas import tpu_sc as plsc

GROUP = 128
NUM_CORES = 2
NUM_SUBCORES = 16
NUM_WORKERS = NUM_CORES * NUM_SUBCORES
LANES = 16
VPAD = 32


@functools.lru_cache(maxsize=None)
def _build_sc_kernel(S0, S1, V, D):
    rows_per_w = S0 // NUM_WORKERS
    groups_per_w = rows_per_w * (S1 // GROUP)
    slab = S1 * D
    chunks = S1 // LANES
    cpg = GROUP // LANES
    mesh = plsc.VectorSubcoreMesh(
        core_axis_name="c", subcore_axis_name="s",
        num_cores=NUM_CORES, num_subcores=NUM_SUBCORES)

    @functools.partial(
        pl.kernel,
        out_type=jax.ShapeDtypeStruct((S0, slab), jnp.float32),
        mesh=mesh,
        compiler_params=pltpu.CompilerParams(
            needs_layout_passes=False, use_tc_tiling_on_sc=False),
        scratch_types=(
            pltpu.VMEM((groups_per_w, GROUP), jnp.int32),
            pltpu.VMEM((VPAD, D), jnp.float32),
            pltpu.VMEM((D * VPAD * LANES,), jnp.float32),
            pltpu.VMEM((slab,), jnp.float32),
            pltpu.VMEM((slab,), jnp.float32),
            pltpu.SemaphoreType.DMA,
            pltpu.SemaphoreType.DMA,
        ),
    )
    def sc_fill(table_hbm, idx_hbm, out_hbm, idx_v, table_v, tableT, buf0,
                buf1, sem0, sem1):
        sid = lax.axis_index("s")
        wid = sid * NUM_CORES + lax.axis_index("c")
        i0 = wid * rows_per_w

        pltpu.sync_copy(idx_hbm.at[pl.ds(wid * groups_per_w, groups_per_w)],
                        idx_v)

        pltpu.sync_copy(table_hbm, table_v.at[pl.ds(0, V)])
        diag0 = lax.iota(jnp.int32, LANES)
        diag1 = diag0 + LANES
        ones = jnp.ones((LANES,), jnp.float32)
        plsc.addupdate_scatter(table_v, [diag0, diag0], ones)
        plsc.addupdate_scatter(table_v, [diag1, diag1], ones,
                               mask=diag1 < V)

        iota16 = lax.iota(jnp.int32, LANES)

        def tput(v, _):
            for dc in range(D // LANES):
                row = table_v[v, pl.dslice(LANES * dc, LANES)]
                for j in range(LANES):
                    d = LANES * dc + j
                    splat = jnp.broadcast_to(row[j], (LANES,))
                    tableT[pl.dslice(VPAD * LANES * d + LANES * v,
                                     LANES)] = splat
            return 0

        lax.fori_loop(0, V, tput, 0)

        bufs = (buf0, buf1)
        sems = (sem0, sem1)

        def fill_row(ii, buf):
            ilv = 4

            def chunk_body(cg, _):
                toks = []
                tjs = []
                for u in range(ilv):
                    ch = cg * ilv + u
                    jt = ch // cpg
                    qq = ch - jt * cpg
                    tokv = idx_v[(S1 // GROUP) * ii + jt,
                                 pl.dslice(qq * LANES, LANES)]
                    toks.append(tokv * LANES + iota16)
                    tjs.append(jt * (8 * GROUP) + qq * LANES)
                for d in range(D):
                    off = (d // 8) * ((S1 // GROUP) * 8 * GROUP) \
                        + (d % 8) * GROUP
                    vs = [plsc.load_gather(
                              tableT.at[pl.ds(VPAD * LANES * d,
                                              VPAD * LANES)], [toks[u]])
                          for u in range(ilv)]
                    for u in range(ilv):
                        buf[pl.ds(tjs[u] + off, LANES)] = vs[u]
                return 0

            lax.fori_loop(0, chunks // ilv, chunk_body, 0)

        def wb(ii, buf, sem):
            return pltpu.make_async_copy(buf, out_hbm.at[i0 + ii], sem)

        def row_pair(p, _):
            ii_a = p * 2
            ii_b = ii_a + 1

            @pl.when(p > 0)
            def _():
                wb(ii_a - 2, buf0, sem0).wait()
            fill_row(ii_a, buf0)
            wb(ii_a, buf0, sem0).start()

            @pl.when(p > 0)
            def _():
                wb(ii_b - 2, buf1, sem1).wait()
            fill_row(ii_b, buf1)
            wb(ii_b, buf1, sem1).start()
            return 0

        lax.fori_loop(0, rows_per_w // 2, row_pair, 0)
        wb(rows_per_w - 2, buf0, sem0).wait()
        wb(rows_per_w - 1, buf1, sem1).wait()

    return sc_fill


def kernel(input, emb_table):
    S0, S1 = input.shape
    V, D = emb_table.shape
    idx2d = input.reshape((S0 * S1) // GROUP, GROUP)
    out = _build_sc_kernel(S0, S1, V, D)(emb_table, idx2d)
    out = out.reshape(S0, D // 8, S1 // GROUP, 8, GROUP)
    out = out.transpose(0, 2, 4, 1, 3).reshape(S0, S1, D)
    return out


# speedup vs baseline: 1.0707x; 1.0090x over previous
import functools

import jax
import jax.numpy as jnp
from jax import lax
from jax.experimental import pallas as pl
from jax.experimental.pallas import tpu as pltpu
from jax.experimental.pallas import tpu_sc as plsc

GROUP = 128
NUM_CORES = 2
NUM_SUBCORES = 16
NUM_WORKERS = NUM_CORES * NUM_SUBCORES
LANES = 16
VPAD = 32


@functools.lru_cache(maxsize=None)
def _build_sc_kernel(S0, S1, V, D):
    rows_per_w = S0 // NUM_WORKERS
    groups_per_w = rows_per_w * (S1 // GROUP)
    slab = S1 * D
    chunks = S1 // LANES
    cpg = GROUP // LANES
    mesh = plsc.VectorSubcoreMesh(
        core_axis_name="c", subcore_axis_name="s",
        num_cores=NUM_CORES, num_subcores=NUM_SUBCORES)

    @functools.partial(
        pl.kernel,
        out_type=jax.ShapeDtypeStruct((S0, slab), jnp.float32),
        mesh=mesh,
        compiler_params=pltpu.CompilerParams(
            needs_layout_passes=False, use_tc_tiling_on_sc=False),
        scratch_types=(
            pltpu.VMEM((rows_per_w // 8, S1 // GROUP, 8, GROUP), jnp.int32),
            pltpu.VMEM((VPAD, D), jnp.float32),
            pltpu.VMEM((D * VPAD * LANES,), jnp.float32),
            pltpu.VMEM((slab,), jnp.float32),
            pltpu.VMEM((slab,), jnp.float32),
            pltpu.SemaphoreType.DMA,
            pltpu.SemaphoreType.DMA,
        ),
    )
    def sc_fill(table_hbm, idx_hbm, out_hbm, idx_v, table_v, tableT, buf0,
                buf1, sem0, sem1):
        sid = lax.axis_index("s")
        wid = sid * NUM_CORES + lax.axis_index("c")
        i0 = wid * rows_per_w

        pltpu.sync_copy(
            idx_hbm.at[pl.ds(wid * (rows_per_w // 8), rows_per_w // 8)],
            idx_v)

        pltpu.sync_copy(table_hbm, table_v.at[pl.ds(0, V)])
        diag0 = lax.iota(jnp.int32, LANES)
        diag1 = diag0 + LANES
        ones = jnp.ones((LANES,), jnp.float32)
        plsc.addupdate_scatter(table_v, [diag0, diag0], ones)
        plsc.addupdate_scatter(table_v, [diag1, diag1], ones,
                               mask=diag1 < V)

        iota16 = lax.iota(jnp.int32, LANES)

        def tput(v, _):
            for dc in range(D // LANES):
                row = table_v[v, pl.dslice(LANES * dc, LANES)]
                for j in range(LANES):
                    d = LANES * dc + j
                    splat = jnp.broadcast_to(row[j], (LANES,))
                    tableT[pl.dslice(VPAD * LANES * d + LANES * v,
                                     LANES)] = splat
            return 0

        lax.fori_loop(0, V, tput, 0)

        bufs = (buf0, buf1)
        sems = (sem0, sem1)

        def fill_row(ii, buf):
            ilv = 4

            def chunk_body(cg, _):
                toks = []
                tjs = []
                it = ii // 8
                ir = ii - it * 8
                for u in range(ilv):
                    ch = cg * ilv + u
                    jt = ch // cpg
                    qq = ch - jt * cpg
                    tokv = idx_v[it, jt, ir, pl.dslice(qq * LANES, LANES)]
                    toks.append(tokv * LANES + iota16)
                    tjs.append(jt * (8 * GROUP) + qq * LANES)
                for d in range(D):
                    off = (d // 8) * ((S1 // GROUP) * 8 * GROUP) \
                        + (d % 8) * GROUP
                    vs = [plsc.load_gather(
                              tableT.at[pl.ds(VPAD * LANES * d,
                                              VPAD * LANES)], [toks[u]])
                          for u in range(ilv)]
                    for u in range(ilv):
                        buf[pl.ds(tjs[u] + off, LANES)] = vs[u]
                return 0

            lax.fori_loop(0, chunks // ilv, chunk_body, 0)

        def wb(ii, buf, sem):
            return pltpu.make_async_copy(buf, out_hbm.at[i0 + ii], sem)

        def row_pair(p, _):
            ii_a = p * 2
            ii_b = ii_a + 1

            @pl.when(p > 0)
            def _():
                wb(ii_a - 2, buf0, sem0).wait()
            fill_row(ii_a, buf0)
            wb(ii_a, buf0, sem0).start()

            @pl.when(p > 0)
            def _():
                wb(ii_b - 2, buf1, sem1).wait()
            fill_row(ii_b, buf1)
            wb(ii_b, buf1, sem1).start()
            return 0

        lax.fori_loop(0, rows_per_w // 2, row_pair, 0)
        wb(rows_per_w - 2, buf0, sem0).wait()
        wb(rows_per_w - 1, buf1, sem1).wait()

    return sc_fill


def kernel(input, emb_table):
    S0, S1 = input.shape
    V, D = emb_table.shape
    idx4 = input.reshape(S0 // 8, 8, S1 // GROUP, GROUP).transpose(0, 2, 1, 3)
    out = _build_sc_kernel(S0, S1, V, D)(emb_table, idx4)
    out = out.reshape(S0, D // 8, S1 // GROUP, 8, GROUP)
    out = out.transpose(0, 2, 4, 1, 3).reshape(S0, S1, D)
    return out
